# baseline (speedup 1.0000x reference)
import functools

import jax
import jax.numpy as jnp
from jax import lax
from jax.experimental import pallas as pl
from jax.experimental.pallas import tpu as pltpu
from jax.experimental.pallas import tpu_sc as plsc

_N = 10000
_E = 320000
_D = 128
_H = 8
_BLK = 2560
_SHIFT = 74.5


def _edge_body(c_ref, ni_ref, nj_ref, p_ref, contrib_ref):
    c = c_ref[0]
    sqrt_c = jnp.sqrt(c)
    ni = ni_ref[...]
    nj = nj_ref[...]
    mni = -ni
    x2 = jnp.sum(mni * mni, axis=-1, keepdims=True)
    y2 = jnp.sum(nj * nj, axis=-1, keepdims=True)
    xy = jnp.sum(mni * nj, axis=-1, keepdims=True)
    num = (1.0 + 2.0 * c * xy + c * y2) * mni + (1.0 - c * x2) * nj
    den = 1.0 + 2.0 * c * xy + c * c * x2 * y2
    ma = num / jnp.maximum(den, 1e-10)
    norm = jnp.sqrt(jnp.sum(ma * ma, axis=-1, keepdims=True))
    norm = jnp.minimum(norm, (1.0 - 1e-5) / sqrt_c)
    zz = sqrt_c * norm
    atanh = 0.5 * (jnp.log1p(zz) - jnp.log1p(-zz))
    dist = (2.0 / sqrt_c) * atanh
    att = -jnp.square(dist)
    p = jnp.exp(att + _SHIFT)
    ny = jnp.sqrt(y2)
    z2 = jnp.minimum(sqrt_c * ny, 1.0 - 1e-7)
    at2 = 0.5 * (jnp.log1p(z2) - jnp.log1p(-z2))
    g = jnp.where(y2 > 0.0, at2 / jnp.maximum(sqrt_c * ny, 1e-30), 1.0)
    p_ref[...] = jnp.broadcast_to(p, (p.shape[0], 8))
    contrib_ref[...] = (p * g) * nj


@jax.jit
def _edge_chain(ni, nj, c):
    nb = _E // _BLK
    p, contrib = pl.pallas_call(
        _edge_body,
        grid=(nb,),
        in_specs=[
            pl.BlockSpec(memory_space=pltpu.SMEM),
            pl.BlockSpec((_BLK, _D), lambda i: (i, 0)),
            pl.BlockSpec((_BLK, _D), lambda i: (i, 0)),
        ],
        out_specs=[
            pl.BlockSpec((_BLK, 8), lambda i: (i, 0)),
            pl.BlockSpec((_BLK, _D), lambda i: (i, 0)),
        ],
        out_shape=[
            jax.ShapeDtypeStruct((_E, 8), jnp.float32),
            jax.ShapeDtypeStruct((_E, _D), jnp.float32),
        ],
    )(c, ni, nj)
    return p, contrib


def _xform_body(c_ref, x_ref, w_ref, th_ref):
    c = c_ref[0]
    sqrt_c = jnp.sqrt(c)
    t = jnp.dot(x_ref[...], w_ref[...], preferred_element_type=jnp.float32)
    nrm = jnp.sqrt(jnp.sum(t * t, axis=-1, keepdims=True))
    nrm = jnp.maximum(nrm, 1e-10)
    th = jnp.tanh(sqrt_c * nrm) * t / (sqrt_c * nrm)
    th_ref[...] = th


@jax.jit
def _xform(x, w, c):
    nblk = 2000
    th = pl.pallas_call(
        _xform_body,
        grid=(_N // nblk,),
        in_specs=[
            pl.BlockSpec(memory_space=pltpu.SMEM),
            pl.BlockSpec((nblk, _D), lambda i: (i, 0)),
            pl.BlockSpec((_D, _D), lambda i: (0, 0)),
        ],
        out_specs=pl.BlockSpec((nblk, _D), lambda i: (i, 0)),
        out_shape=jax.ShapeDtypeStruct((_N, _D), jnp.float32),
    )(c, x, w)
    return th


_NW = 32
_EPW = _E // _NW
_GK = 200


def _gather_body(th_hbm, row_hbm, col_hbm, ni_hbm, nj_hbm,
                 idxr0, idxc0, idxr1, idxc1, bni0, bnj0, bni1, bnj1,
                 semni0, semnj0, semni1, semnj1):
    wid = lax.axis_index("s") * 2 + lax.axis_index("c")
    base = wid * _EPW
    npairs = _EPW // (2 * _GK)

    def load_idx(off, ir, ic):
        pltpu.sync_copy(row_hbm.at[pl.ds(off, _GK)], ir)
        pltpu.sync_copy(col_hbm.at[pl.ds(off, _GK)], ic)

    def fire(ir, ic, bn, bj, sn, sj):
        pltpu.async_copy(th_hbm.at[ir], bn, sn)
        pltpu.async_copy(th_hbm.at[ic], bj, sj)

    def drain_write(off, ir, ic, bn, bj, sn, sj):
        pltpu.make_async_copy(th_hbm.at[ir], bn, sn).wait()
        pltpu.make_async_copy(th_hbm.at[ic], bj, sj).wait()
        pltpu.sync_copy(bn, ni_hbm.at[pl.ds(off, _GK)])
        pltpu.sync_copy(bj, nj_hbm.at[pl.ds(off, _GK)])

    load_idx(base, idxr0, idxc0)
    fire(idxr0, idxc0, bni0, bnj0, semni0, semnj0)

    def pair(j, carry):
        off0 = base + (2 * j) * _GK
        off1 = off0 + _GK
        load_idx(off1, idxr1, idxc1)
        fire(idxr1, idxc1, bni1, bnj1, semni1, semnj1)
        drain_write(off0, idxr0, idxc0, bni0, bnj0, semni0, semnj0)
        @pl.when(j < npairs - 1)
        def _():
            load_idx(off1 + _GK, idxr0, idxc0)
            fire(idxr0, idxc0, bni0, bnj0, semni0, semnj0)
        drain_write(off1, idxr1, idxc1, bni1, bnj1, semni1, semnj1)
        return carry

    lax.fori_loop(0, npairs, pair, 0)


@jax.jit
def _gather(th, row, col):
    mesh = plsc.VectorSubcoreMesh(core_axis_name="c", subcore_axis_name="s")
    f = functools.partial(
        pl.kernel,
        out_type=[
            jax.ShapeDtypeStruct((_E, _D), jnp.float32),
            jax.ShapeDtypeStruct((_E, _D), jnp.float32),
        ],
        mesh=mesh,
        scratch_types=[
            pltpu.VMEM((_GK,), jnp.int32),
            pltpu.VMEM((_GK,), jnp.int32),
            pltpu.VMEM((_GK,), jnp.int32),
            pltpu.VMEM((_GK,), jnp.int32),
            pltpu.VMEM((_GK, _D), jnp.float32),
            pltpu.VMEM((_GK, _D), jnp.float32),
            pltpu.VMEM((_GK, _D), jnp.float32),
            pltpu.VMEM((_GK, _D), jnp.float32),
            pltpu.SemaphoreType.DMA,
            pltpu.SemaphoreType.DMA,
            pltpu.SemaphoreType.DMA,
            pltpu.SemaphoreType.DMA,
        ],
    )(_gather_body)
    return f(th, row, col)


_SK = 128
_STAIL = _EPW - (_EPW // _SK) * _SK
_NPAD = 10240
_STRIPE = _NPAD // 16


def _scatter_body(p8_hbm, c_hbm, row_hbm, zm_hbm, zd_hbm, den2_hbm, msg2_hbm,
                  idx, pb, cb, idxt, pbt, cbt, acc_den, acc_msg):
    cid = lax.axis_index("c")
    sid = lax.axis_index("s")
    wid = sid * 2 + cid
    base = wid * _EPW
    r0 = sid * _STRIPE
    pltpu.sync_copy(zm_hbm, acc_msg.at[pl.ds(r0, _STRIPE)])
    pltpu.sync_copy(zd_hbm, acc_den.at[pl.ds(r0, _STRIPE)])
    plsc.subcore_barrier()

    def step(i, carry):
        off = base + i * _SK
        pltpu.sync_copy(row_hbm.at[pl.ds(off, _SK)], idx)
        pltpu.sync_copy(p8_hbm.at[pl.ds(off, _SK)], pb)
        pltpu.sync_copy(c_hbm.at[pl.ds(off, _SK)], cb)
        pltpu.sync_copy(pb, acc_den.at[idx], add=True)
        pltpu.sync_copy(cb, acc_msg.at[idx], add=True)
        return carry

    lax.fori_loop(0, _EPW // _SK, step, 0)
    toff = base + (_EPW // _SK) * _SK
    pltpu.sync_copy(row_hbm.at[pl.ds(toff, _STAIL)], idxt)
    pltpu.sync_copy(p8_hbm.at[pl.ds(toff, _STAIL)], pbt)
    pltpu.sync_copy(c_hbm.at[pl.ds(toff, _STAIL)], cbt)
    pltpu.sync_copy(pbt, acc_den.at[idxt], add=True)
    pltpu.sync_copy(cbt, acc_msg.at[idxt], add=True)
    plsc.subcore_barrier()
    pltpu.sync_copy(acc_den.at[pl.ds(r0, _STRIPE)],
                    den2_hbm.at[cid, pl.ds(r0, _STRIPE)])
    pltpu.sync_copy(acc_msg.at[pl.ds(r0, _STRIPE)],
                    msg2_hbm.at[cid, pl.ds(r0, _STRIPE)])


@jax.jit
def _scatter(p8, contrib, row, zm, zd):
    mesh = plsc.VectorSubcoreMesh(core_axis_name="c", subcore_axis_name="s")
    f = functools.partial(
        pl.kernel,
        out_type=[
            jax.ShapeDtypeStruct((2, _NPAD, 8), jnp.float32),
            jax.ShapeDtypeStruct((2, _NPAD, _D), jnp.float32),
        ],
        mesh=mesh,
        scratch_types=[
            pltpu.VMEM((_SK,), jnp.int32),
            pltpu.VMEM((_SK, 8), jnp.float32),
            pltpu.VMEM((_SK, _D), jnp.float32),
            pltpu.VMEM((_STAIL,), jnp.int32),
            pltpu.VMEM((_STAIL, 8), jnp.float32),
            pltpu.VMEM((_STAIL, _D), jnp.float32),
            pltpu.VMEM_SHARED((_NPAD, 8), jnp.float32),
            pltpu.VMEM_SHARED((_NPAD, _D), jnp.float32),
        ],
    )(_scatter_body)
    return f(p8, contrib, row, zm, zd)


def _epilogue_body(eps_ref, den2_ref, msg2_ref, x_ref, sw_ref, w1_ref, b1_ref,
                   w2_ref, b2_ref, g1_ref, be1_ref, g2_ref, be2_ref, acc_ref,
                   out_ref):
    eps = eps_ref[0]
    denom = den2_ref[0, :_N, 0:1] + den2_ref[1, :_N, 0:1]
    msg = msg2_ref[0, :_N, :] + msg2_ref[1, :_N, :]
    messages = msg / jnp.maximum(denom, 1e-37)
    self_t = jnp.dot(x_ref[...], sw_ref[...], preferred_element_type=jnp.float32)
    out = messages + (1.0 + eps) * self_t
    m1 = jnp.mean(out, axis=0, keepdims=True)
    v1 = jnp.mean(jnp.square(out - m1), axis=0, keepdims=True)
    out = g1_ref[...] * (out - m1) / jnp.sqrt(v1 + 1e-3) + be1_ref[...]
    hid = jnp.dot(out, w1_ref[...], preferred_element_type=jnp.float32) + b1_ref[...]
    hid = jnp.where(hid >= 0.0, hid, 0.1 * hid)
    m2 = jnp.mean(hid, axis=0, keepdims=True)
    v2 = jnp.mean(jnp.square(hid - m2), axis=0, keepdims=True)
    hid = g2_ref[...] * (hid - m2) / jnp.sqrt(v2 + 1e-3) + be2_ref[...]
    of = jnp.dot(hid, w2_ref[...], preferred_element_type=jnp.float32) + b2_ref[...]
    of = jnp.where(of > 0.0, of, jnp.exp(of) - 1.0)
    out_ref[...] = acc_ref[...] + of


@jax.jit
def _epilogue(den2, msg2, x, sw, w1, b1, w2, b2, g1, be1, g2, be2, acc, eps):
    return pl.pallas_call(
        _epilogue_body,
        in_specs=[pl.BlockSpec(memory_space=pltpu.SMEM)] + [pl.BlockSpec()] * 13,
        out_specs=pl.BlockSpec(),
        out_shape=jax.ShapeDtypeStruct((_N, _D), jnp.float32),
    )(eps.reshape(1), den2, msg2, x, sw, w1, b1.reshape(1, -1), w2,
      b2.reshape(1, -1), g1.reshape(1, -1), be1.reshape(1, -1),
      g2.reshape(1, -1), be2.reshape(1, -1), acc)


def kernel(x, adj_indices, transform, self_weight, mlp_w1, mlp_b1, mlp_w2,
           mlp_b2, bn1_gamma, bn1_beta, bn2_gamma, bn2_beta, curvature, epsilon):
    row = adj_indices[:, 0]
    col = adj_indices[:, 1]
    zm = jnp.zeros((_STRIPE, _D), dtype=jnp.float32)
    zd = jnp.zeros((_STRIPE, 8), dtype=jnp.float32)
    acc = jnp.zeros((_N, _D), dtype=jnp.float32)
    for h in range(_H):
        th = _xform(x, transform[h], curvature)
        ni, nj = _gather(th, row, col)
        p8, contrib = _edge_chain(ni, nj, curvature)
        den2, msg2 = _scatter(p8, contrib, row, zm, zd)
        acc = _epilogue(den2, msg2, x, self_weight[h], mlp_w1, mlp_b1,
                        mlp_w2, mlp_b2, bn1_gamma, bn1_beta, bn2_gamma,
                        bn2_beta, acc, epsilon)
    output = acc / 8.0
    return (output, curvature)


# speedup vs baseline: 1.1092x; 1.1092x over previous
import functools

import jax
import jax.numpy as jnp
from jax import lax
from jax.experimental import pallas as pl
from jax.experimental.pallas import tpu as pltpu
from jax.experimental.pallas import tpu_sc as plsc

_N = 10000
_E = 320000
_D = 128
_H = 8
_BLK = 2560
_SHIFT = 74.5


def _edge_body(c_ref, ni_ref, nj_ref, p_ref, contrib_ref):
    c = c_ref[0]
    sqrt_c = jnp.sqrt(c)
    ni = ni_ref[...]
    nj = nj_ref[...]
    mni = -ni
    x2 = jnp.sum(mni * mni, axis=-1, keepdims=True)
    y2 = jnp.sum(nj * nj, axis=-1, keepdims=True)
    xy = jnp.sum(mni * nj, axis=-1, keepdims=True)
    num = (1.0 + 2.0 * c * xy + c * y2) * mni + (1.0 - c * x2) * nj
    den = 1.0 + 2.0 * c * xy + c * c * x2 * y2
    ma = num / jnp.maximum(den, 1e-10)
    norm = jnp.sqrt(jnp.sum(ma * ma, axis=-1, keepdims=True))
    norm = jnp.minimum(norm, (1.0 - 1e-5) / sqrt_c)
    zz = sqrt_c * norm
    atanh = 0.5 * (jnp.log1p(zz) - jnp.log1p(-zz))
    dist = (2.0 / sqrt_c) * atanh
    att = -jnp.square(dist)
    p = jnp.exp(att + _SHIFT)
    ny = jnp.sqrt(y2)
    z2 = jnp.minimum(sqrt_c * ny, 1.0 - 1e-7)
    at2 = 0.5 * (jnp.log1p(z2) - jnp.log1p(-z2))
    g = jnp.where(y2 > 0.0, at2 / jnp.maximum(sqrt_c * ny, 1e-30), 1.0)
    p_ref[...] = jnp.broadcast_to(p, (p.shape[0], 8))
    contrib_ref[...] = (p * g) * nj


@jax.jit
def _edge_chain(ni, nj, c):
    nb = _E // _BLK
    p, contrib = pl.pallas_call(
        _edge_body,
        grid=(nb,),
        in_specs=[
            pl.BlockSpec(memory_space=pltpu.SMEM),
            pl.BlockSpec((_BLK, _D), lambda i: (i, 0)),
            pl.BlockSpec((_BLK, _D), lambda i: (i, 0)),
        ],
        out_specs=[
            pl.BlockSpec((_BLK, 8), lambda i: (i, 0)),
            pl.BlockSpec((_BLK, _D), lambda i: (i, 0)),
        ],
        out_shape=[
            jax.ShapeDtypeStruct((_E, 8), jnp.float32),
            jax.ShapeDtypeStruct((_E, _D), jnp.float32),
        ],
    )(c, ni, nj)
    return p, contrib


def _xform_body(c_ref, x_ref, w_ref, th_ref):
    c = c_ref[0]
    sqrt_c = jnp.sqrt(c)
    t = jnp.dot(x_ref[...], w_ref[...], preferred_element_type=jnp.float32)
    nrm = jnp.sqrt(jnp.sum(t * t, axis=-1, keepdims=True))
    nrm = jnp.maximum(nrm, 1e-10)
    th = jnp.tanh(sqrt_c * nrm) * t / (sqrt_c * nrm)
    th_ref[...] = th


@jax.jit
def _xform(x, w, c):
    nblk = 2000
    th = pl.pallas_call(
        _xform_body,
        grid=(_N // nblk,),
        in_specs=[
            pl.BlockSpec(memory_space=pltpu.SMEM),
            pl.BlockSpec((nblk, _D), lambda i: (i, 0)),
            pl.BlockSpec((_D, _D), lambda i: (0, 0)),
        ],
        out_specs=pl.BlockSpec((nblk, _D), lambda i: (i, 0)),
        out_shape=jax.ShapeDtypeStruct((_N, _D), jnp.float32),
    )(c, x, w)
    return th


_NW = 32
_EPW = _E // _NW
_GK = 200


def _gather_body(th_hbm, row_hbm, col_hbm, ni_hbm, nj_hbm,
                 idxr0, idxc0, idxr1, idxc1, bni0, bnj0, bni1, bnj1,
                 semni0, semnj0, semni1, semnj1):
    wid = lax.axis_index("s") * 2 + lax.axis_index("c")
    base = wid * _EPW
    npairs = _EPW // (2 * _GK)

    def load_idx(off, ir, ic):
        pltpu.sync_copy(row_hbm.at[pl.ds(off, _GK)], ir)
        pltpu.sync_copy(col_hbm.at[pl.ds(off, _GK)], ic)

    def fire(ir, ic, bn, bj, sn, sj):
        pltpu.async_copy(th_hbm.at[ir], bn, sn)
        pltpu.async_copy(th_hbm.at[ic], bj, sj)

    def drain_write(off, ir, ic, bn, bj, sn, sj):
        pltpu.make_async_copy(th_hbm.at[ir], bn, sn).wait()
        pltpu.make_async_copy(th_hbm.at[ic], bj, sj).wait()
        pltpu.sync_copy(bn, ni_hbm.at[pl.ds(off, _GK)])
        pltpu.sync_copy(bj, nj_hbm.at[pl.ds(off, _GK)])

    load_idx(base, idxr0, idxc0)
    fire(idxr0, idxc0, bni0, bnj0, semni0, semnj0)

    def pair(j, carry):
        off0 = base + (2 * j) * _GK
        off1 = off0 + _GK
        load_idx(off1, idxr1, idxc1)
        fire(idxr1, idxc1, bni1, bnj1, semni1, semnj1)
        drain_write(off0, idxr0, idxc0, bni0, bnj0, semni0, semnj0)
        @pl.when(j < npairs - 1)
        def _():
            load_idx(off1 + _GK, idxr0, idxc0)
            fire(idxr0, idxc0, bni0, bnj0, semni0, semnj0)
        drain_write(off1, idxr1, idxc1, bni1, bnj1, semni1, semnj1)
        return carry

    lax.fori_loop(0, npairs, pair, 0)


@jax.jit
def _gather(th, row, col):
    mesh = plsc.VectorSubcoreMesh(core_axis_name="c", subcore_axis_name="s")
    f = functools.partial(
        pl.kernel,
        out_type=[
            jax.ShapeDtypeStruct((_E, _D), jnp.float32),
            jax.ShapeDtypeStruct((_E, _D), jnp.float32),
        ],
        mesh=mesh,
        scratch_types=[
            pltpu.VMEM((_GK,), jnp.int32),
            pltpu.VMEM((_GK,), jnp.int32),
            pltpu.VMEM((_GK,), jnp.int32),
            pltpu.VMEM((_GK,), jnp.int32),
            pltpu.VMEM((_GK, _D), jnp.float32),
            pltpu.VMEM((_GK, _D), jnp.float32),
            pltpu.VMEM((_GK, _D), jnp.float32),
            pltpu.VMEM((_GK, _D), jnp.float32),
            pltpu.SemaphoreType.DMA,
            pltpu.SemaphoreType.DMA,
            pltpu.SemaphoreType.DMA,
            pltpu.SemaphoreType.DMA,
        ],
    )(_gather_body)
    return f(th, row, col)


_SK = 64
_STAIL = _EPW - (_EPW // _SK) * _SK
_NPAD = 10240
_STRIPE = _NPAD // 16


def _scatter_body(p8_hbm, c_hbm, row_hbm, zm_hbm, zd_hbm, den2_hbm, msg2_hbm,
                  idx0, pb0, cb0, idx1, pb1, cb1, idxt, pbt, cbt,
                  si0, sp0, sc0, si1, sp1, sc1, acc_den, acc_msg):
    cid = lax.axis_index("c")
    sid = lax.axis_index("s")
    wid = sid * 2 + cid
    base = wid * _EPW
    r0 = sid * _STRIPE
    pltpu.sync_copy(zm_hbm, acc_msg.at[pl.ds(r0, _STRIPE)])
    pltpu.sync_copy(zd_hbm, acc_den.at[pl.ds(r0, _STRIPE)])
    plsc.subcore_barrier()
    nchunks = _EPW // _SK
    npairs = nchunks // 2

    def fire(off, ix, pb, cb, s1, s2, s3):
        pltpu.async_copy(row_hbm.at[pl.ds(off, _SK)], ix, s1)
        pltpu.async_copy(p8_hbm.at[pl.ds(off, _SK)], pb, s2)
        pltpu.async_copy(c_hbm.at[pl.ds(off, _SK)], cb, s3)

    def drain_scatter(off, ix, pb, cb, s1, s2, s3):
        pltpu.make_async_copy(row_hbm.at[pl.ds(off, _SK)], ix, s1).wait()
        pltpu.make_async_copy(p8_hbm.at[pl.ds(off, _SK)], pb, s2).wait()
        pltpu.make_async_copy(c_hbm.at[pl.ds(off, _SK)], cb, s3).wait()
        pltpu.sync_copy(pb, acc_den.at[ix], add=True)
        pltpu.sync_copy(cb, acc_msg.at[ix], add=True)

    fire(base, idx0, pb0, cb0, si0, sp0, sc0)

    def pair(j, carry):
        off0 = base + (2 * j) * _SK
        off1 = off0 + _SK
        fire(off1, idx1, pb1, cb1, si1, sp1, sc1)
        drain_scatter(off0, idx0, pb0, cb0, si0, sp0, sc0)

        @pl.when(j < npairs - 1)
        def _():
            fire(off1 + _SK, idx0, pb0, cb0, si0, sp0, sc0)
        drain_scatter(off1, idx1, pb1, cb1, si1, sp1, sc1)
        return carry

    lax.fori_loop(0, npairs, pair, 0)
    toff = base + (_EPW // _SK) * _SK
    pltpu.sync_copy(row_hbm.at[pl.ds(toff, _STAIL)], idxt)
    pltpu.sync_copy(p8_hbm.at[pl.ds(toff, _STAIL)], pbt)
    pltpu.sync_copy(c_hbm.at[pl.ds(toff, _STAIL)], cbt)
    pltpu.sync_copy(pbt, acc_den.at[idxt], add=True)
    pltpu.sync_copy(cbt, acc_msg.at[idxt], add=True)
    plsc.subcore_barrier()
    pltpu.sync_copy(acc_den.at[pl.ds(r0, _STRIPE)],
                    den2_hbm.at[cid, pl.ds(r0, _STRIPE)])
    pltpu.sync_copy(acc_msg.at[pl.ds(r0, _STRIPE)],
                    msg2_hbm.at[cid, pl.ds(r0, _STRIPE)])


@jax.jit
def _scatter(p8, contrib, row, zm, zd):
    mesh = plsc.VectorSubcoreMesh(core_axis_name="c", subcore_axis_name="s")
    f = functools.partial(
        pl.kernel,
        out_type=[
            jax.ShapeDtypeStruct((2, _NPAD, 8), jnp.float32),
            jax.ShapeDtypeStruct((2, _NPAD, _D), jnp.float32),
        ],
        mesh=mesh,
        scratch_types=[
            pltpu.VMEM((_SK,), jnp.int32),
            pltpu.VMEM((_SK, 8), jnp.float32),
            pltpu.VMEM((_SK, _D), jnp.float32),
            pltpu.VMEM((_SK,), jnp.int32),
            pltpu.VMEM((_SK, 8), jnp.float32),
            pltpu.VMEM((_SK, _D), jnp.float32),
            pltpu.VMEM((_STAIL,), jnp.int32),
            pltpu.VMEM((_STAIL, 8), jnp.float32),
            pltpu.VMEM((_STAIL, _D), jnp.float32),
            pltpu.SemaphoreType.DMA,
            pltpu.SemaphoreType.DMA,
            pltpu.SemaphoreType.DMA,
            pltpu.SemaphoreType.DMA,
            pltpu.SemaphoreType.DMA,
            pltpu.SemaphoreType.DMA,
            pltpu.VMEM_SHARED((_NPAD, 8), jnp.float32),
            pltpu.VMEM_SHARED((_NPAD, _D), jnp.float32),
        ],
    )(_scatter_body)
    return f(p8, contrib, row, zm, zd)


def _epilogue_body(eps_ref, den2_ref, msg2_ref, x_ref, sw_ref, w1_ref, b1_ref,
                   w2_ref, b2_ref, g1_ref, be1_ref, g2_ref, be2_ref, acc_ref,
                   out_ref):
    eps = eps_ref[0]
    denom = den2_ref[0, :_N, 0:1] + den2_ref[1, :_N, 0:1]
    msg = msg2_ref[0, :_N, :] + msg2_ref[1, :_N, :]
    messages = msg / jnp.maximum(denom, 1e-37)
    self_t = jnp.dot(x_ref[...], sw_ref[...], preferred_element_type=jnp.float32)
    out = messages + (1.0 + eps) * self_t
    m1 = jnp.mean(out, axis=0, keepdims=True)
    v1 = jnp.mean(jnp.square(out - m1), axis=0, keepdims=True)
    out = g1_ref[...] * (out - m1) / jnp.sqrt(v1 + 1e-3) + be1_ref[...]
    hid = jnp.dot(out, w1_ref[...], preferred_element_type=jnp.float32) + b1_ref[...]
    hid = jnp.where(hid >= 0.0, hid, 0.1 * hid)
    m2 = jnp.mean(hid, axis=0, keepdims=True)
    v2 = jnp.mean(jnp.square(hid - m2), axis=0, keepdims=True)
    hid = g2_ref[...] * (hid - m2) / jnp.sqrt(v2 + 1e-3) + be2_ref[...]
    of = jnp.dot(hid, w2_ref[...], preferred_element_type=jnp.float32) + b2_ref[...]
    of = jnp.where(of > 0.0, of, jnp.exp(of) - 1.0)
    out_ref[...] = acc_ref[...] + of


@jax.jit
def _epilogue(den2, msg2, x, sw, w1, b1, w2, b2, g1, be1, g2, be2, acc, eps):
    return pl.pallas_call(
        _epilogue_body,
        in_specs=[pl.BlockSpec(memory_space=pltpu.SMEM)] + [pl.BlockSpec()] * 13,
        out_specs=pl.BlockSpec(),
        out_shape=jax.ShapeDtypeStruct((_N, _D), jnp.float32),
    )(eps.reshape(1), den2, msg2, x, sw, w1, b1.reshape(1, -1), w2,
      b2.reshape(1, -1), g1.reshape(1, -1), be1.reshape(1, -1),
      g2.reshape(1, -1), be2.reshape(1, -1), acc)


def kernel(x, adj_indices, transform, self_weight, mlp_w1, mlp_b1, mlp_w2,
           mlp_b2, bn1_gamma, bn1_beta, bn2_gamma, bn2_beta, curvature, epsilon):
    row = adj_indices[:, 0]
    col = adj_indices[:, 1]
    zm = jnp.zeros((_STRIPE, _D), dtype=jnp.float32)
    zd = jnp.zeros((_STRIPE, 8), dtype=jnp.float32)
    acc = jnp.zeros((_N, _D), dtype=jnp.float32)
    for h in range(_H):
        th = _xform(x, transform[h], curvature)
        ni, nj = _gather(th, row, col)
        p8, contrib = _edge_chain(ni, nj, curvature)
        den2, msg2 = _scatter(p8, contrib, row, zm, zd)
        acc = _epilogue(den2, msg2, x, self_weight[h], mlp_w1, mlp_b1,
                        mlp_w2, mlp_b2, bn1_gamma, bn1_beta, bn2_gamma,
                        bn2_beta, acc, epsilon)
    output = acc / 8.0
    return (output, curvature)


# speedup vs baseline: 1.1098x; 1.0005x over previous
import functools

import jax
import jax.numpy as jnp
from jax import lax
from jax.experimental import pallas as pl
from jax.experimental.pallas import tpu as pltpu
from jax.experimental.pallas import tpu_sc as plsc

_N = 10000
_E = 320000
_D = 128
_H = 8
_BLK = 2560
_SHIFT = 74.5


def _edge_body(c_ref, ni_ref, nj_ref, p_ref, contrib_ref):
    c = c_ref[0]
    sqrt_c = jnp.sqrt(c)
    ni = ni_ref[...]
    nj = nj_ref[...]
    mni = -ni
    x2 = jnp.sum(mni * mni, axis=-1, keepdims=True)
    y2 = jnp.sum(nj * nj, axis=-1, keepdims=True)
    xy = jnp.sum(mni * nj, axis=-1, keepdims=True)
    num = (1.0 + 2.0 * c * xy + c * y2) * mni + (1.0 - c * x2) * nj
    den = 1.0 + 2.0 * c * xy + c * c * x2 * y2
    ma = num / jnp.maximum(den, 1e-10)
    norm = jnp.sqrt(jnp.sum(ma * ma, axis=-1, keepdims=True))
    norm = jnp.minimum(norm, (1.0 - 1e-5) / sqrt_c)
    zz = sqrt_c * norm
    atanh = 0.5 * (jnp.log1p(zz) - jnp.log1p(-zz))
    dist = (2.0 / sqrt_c) * atanh
    att = -jnp.square(dist)
    p = jnp.exp(att + _SHIFT)
    ny = jnp.sqrt(y2)
    z2 = jnp.minimum(sqrt_c * ny, 1.0 - 1e-7)
    at2 = 0.5 * (jnp.log1p(z2) - jnp.log1p(-z2))
    g = jnp.where(y2 > 0.0, at2 / jnp.maximum(sqrt_c * ny, 1e-30), 1.0)
    p_ref[...] = jnp.broadcast_to(p, (p.shape[0], 8))
    contrib_ref[...] = (p * g) * nj


@jax.jit
def _edge_chain(ni, nj, c):
    nb = _E // _BLK
    p, contrib = pl.pallas_call(
        _edge_body,
        grid=(nb,),
        in_specs=[
            pl.BlockSpec(memory_space=pltpu.SMEM),
            pl.BlockSpec((_BLK, _D), lambda i: (i, 0)),
            pl.BlockSpec((_BLK, _D), lambda i: (i, 0)),
        ],
        out_specs=[
            pl.BlockSpec((_BLK, 8), lambda i: (i, 0)),
            pl.BlockSpec((_BLK, _D), lambda i: (i, 0)),
        ],
        out_shape=[
            jax.ShapeDtypeStruct((_E, 8), jnp.float32),
            jax.ShapeDtypeStruct((_E, _D), jnp.float32),
        ],
    )(c, ni, nj)
    return p, contrib


def _xform_body(c_ref, x_ref, w_ref, th_ref):
    c = c_ref[0]
    sqrt_c = jnp.sqrt(c)
    t = jnp.dot(x_ref[...], w_ref[...], preferred_element_type=jnp.float32)
    nrm = jnp.sqrt(jnp.sum(t * t, axis=-1, keepdims=True))
    nrm = jnp.maximum(nrm, 1e-10)
    th = jnp.tanh(sqrt_c * nrm) * t / (sqrt_c * nrm)
    th_ref[...] = th


@jax.jit
def _xform(x, w, c):
    nblk = 2000
    th = pl.pallas_call(
        _xform_body,
        grid=(_N // nblk,),
        in_specs=[
            pl.BlockSpec(memory_space=pltpu.SMEM),
            pl.BlockSpec((nblk, _D), lambda i: (i, 0)),
            pl.BlockSpec((_D, _D), lambda i: (0, 0)),
        ],
        out_specs=pl.BlockSpec((nblk, _D), lambda i: (i, 0)),
        out_shape=jax.ShapeDtypeStruct((_N, _D), jnp.float32),
    )(c, x, w)
    return th


_NW = 32
_EPW = _E // _NW
_GK = 200


def _gather_body(th_hbm, row_hbm, col_hbm, ni_hbm, nj_hbm,
                 idxr, idxc, bni0, bnj0, bni1, bnj1,
                 semni0, semnj0, semni1, semnj1):
    wid = lax.axis_index("s") * 2 + lax.axis_index("c")
    base = wid * _EPW
    npairs = _EPW // (2 * _GK)
    pltpu.sync_copy(row_hbm.at[pl.ds(base, _EPW)], idxr)
    pltpu.sync_copy(col_hbm.at[pl.ds(base, _EPW)], idxc)

    def fire(loc, bn, bj, sn, sj):
        pltpu.async_copy(th_hbm.at[idxr.at[pl.ds(loc, _GK)]], bn, sn)
        pltpu.async_copy(th_hbm.at[idxc.at[pl.ds(loc, _GK)]], bj, sj)

    def drain_write(loc, bn, bj, sn, sj):
        pltpu.make_async_copy(th_hbm.at[idxr.at[pl.ds(loc, _GK)]], bn, sn).wait()
        pltpu.make_async_copy(th_hbm.at[idxc.at[pl.ds(loc, _GK)]], bj, sj).wait()
        pltpu.sync_copy(bn, ni_hbm.at[pl.ds(base + loc, _GK)])
        pltpu.sync_copy(bj, nj_hbm.at[pl.ds(base + loc, _GK)])

    fire(0, bni0, bnj0, semni0, semnj0)

    def pair(j, carry):
        loc0 = (2 * j) * _GK
        loc1 = loc0 + _GK
        fire(loc1, bni1, bnj1, semni1, semnj1)
        drain_write(loc0, bni0, bnj0, semni0, semnj0)

        @pl.when(j < npairs - 1)
        def _():
            fire(loc1 + _GK, bni0, bnj0, semni0, semnj0)
        drain_write(loc1, bni1, bnj1, semni1, semnj1)
        return carry

    lax.fori_loop(0, npairs, pair, 0)


@jax.jit
def _gather(th, row, col):
    mesh = plsc.VectorSubcoreMesh(core_axis_name="c", subcore_axis_name="s")
    f = functools.partial(
        pl.kernel,
        out_type=[
            jax.ShapeDtypeStruct((_E, _D), jnp.float32),
            jax.ShapeDtypeStruct((_E, _D), jnp.float32),
        ],
        mesh=mesh,
        scratch_types=[
            pltpu.VMEM((_EPW,), jnp.int32),
            pltpu.VMEM((_EPW,), jnp.int32),
            pltpu.VMEM((_GK, _D), jnp.float32),
            pltpu.VMEM((_GK, _D), jnp.float32),
            pltpu.VMEM((_GK, _D), jnp.float32),
            pltpu.VMEM((_GK, _D), jnp.float32),
            pltpu.SemaphoreType.DMA,
            pltpu.SemaphoreType.DMA,
            pltpu.SemaphoreType.DMA,
            pltpu.SemaphoreType.DMA,
        ],
    )(_gather_body)
    return f(th, row, col)


_SK = 64
_STAIL = _EPW - (_EPW // _SK) * _SK
_NPAD = 10240
_STRIPE = _NPAD // 16


def _scatter_body(p8_hbm, c_hbm, row_hbm, zm_hbm, zd_hbm, den2_hbm, msg2_hbm,
                  idx0, pb0, cb0, idx1, pb1, cb1, idxt, pbt, cbt,
                  si0, sp0, sc0, si1, sp1, sc1, acc_den, acc_msg):
    cid = lax.axis_index("c")
    sid = lax.axis_index("s")
    wid = sid * 2 + cid
    base = wid * _EPW
    r0 = sid * _STRIPE
    pltpu.sync_copy(zm_hbm, acc_msg.at[pl.ds(r0, _STRIPE)])
    pltpu.sync_copy(zd_hbm, acc_den.at[pl.ds(r0, _STRIPE)])
    plsc.subcore_barrier()
    nchunks = _EPW // _SK
    npairs = nchunks // 2

    def fire(off, ix, pb, cb, s1, s2, s3):
        pltpu.async_copy(row_hbm.at[pl.ds(off, _SK)], ix, s1)
        pltpu.async_copy(p8_hbm.at[pl.ds(off, _SK)], pb, s2)
        pltpu.async_copy(c_hbm.at[pl.ds(off, _SK)], cb, s3)

    def drain_scatter(off, ix, pb, cb, s1, s2, s3):
        pltpu.make_async_copy(row_hbm.at[pl.ds(off, _SK)], ix, s1).wait()
        pltpu.make_async_copy(p8_hbm.at[pl.ds(off, _SK)], pb, s2).wait()
        pltpu.make_async_copy(c_hbm.at[pl.ds(off, _SK)], cb, s3).wait()
        pltpu.sync_copy(pb, acc_den.at[ix], add=True)
        pltpu.sync_copy(cb, acc_msg.at[ix], add=True)

    fire(base, idx0, pb0, cb0, si0, sp0, sc0)

    def pair(j, carry):
        off0 = base + (2 * j) * _SK
        off1 = off0 + _SK
        fire(off1, idx1, pb1, cb1, si1, sp1, sc1)
        drain_scatter(off0, idx0, pb0, cb0, si0, sp0, sc0)

        @pl.when(j < npairs - 1)
        def _():
            fire(off1 + _SK, idx0, pb0, cb0, si0, sp0, sc0)
        drain_scatter(off1, idx1, pb1, cb1, si1, sp1, sc1)
        return carry

    lax.fori_loop(0, npairs, pair, 0)
    toff = base + (_EPW // _SK) * _SK
    pltpu.sync_copy(row_hbm.at[pl.ds(toff, _STAIL)], idxt)
    pltpu.sync_copy(p8_hbm.at[pl.ds(toff, _STAIL)], pbt)
    pltpu.sync_copy(c_hbm.at[pl.ds(toff, _STAIL)], cbt)
    pltpu.sync_copy(pbt, acc_den.at[idxt], add=True)
    pltpu.sync_copy(cbt, acc_msg.at[idxt], add=True)
    plsc.subcore_barrier()
    pltpu.sync_copy(acc_den.at[pl.ds(r0, _STRIPE)],
                    den2_hbm.at[cid, pl.ds(r0, _STRIPE)])
    pltpu.sync_copy(acc_msg.at[pl.ds(r0, _STRIPE)],
                    msg2_hbm.at[cid, pl.ds(r0, _STRIPE)])


@jax.jit
def _scatter(p8, contrib, row, zm, zd):
    mesh = plsc.VectorSubcoreMesh(core_axis_name="c", subcore_axis_name="s")
    f = functools.partial(
        pl.kernel,
        out_type=[
            jax.ShapeDtypeStruct((2, _NPAD, 8), jnp.float32),
            jax.ShapeDtypeStruct((2, _NPAD, _D), jnp.float32),
        ],
        mesh=mesh,
        scratch_types=[
            pltpu.VMEM((_SK,), jnp.int32),
            pltpu.VMEM((_SK, 8), jnp.float32),
            pltpu.VMEM((_SK, _D), jnp.float32),
            pltpu.VMEM((_SK,), jnp.int32),
            pltpu.VMEM((_SK, 8), jnp.float32),
            pltpu.VMEM((_SK, _D), jnp.float32),
            pltpu.VMEM((_STAIL,), jnp.int32),
            pltpu.VMEM((_STAIL, 8), jnp.float32),
            pltpu.VMEM((_STAIL, _D), jnp.float32),
            pltpu.SemaphoreType.DMA,
            pltpu.SemaphoreType.DMA,
            pltpu.SemaphoreType.DMA,
            pltpu.SemaphoreType.DMA,
            pltpu.SemaphoreType.DMA,
            pltpu.SemaphoreType.DMA,
            pltpu.VMEM_SHARED((_NPAD, 8), jnp.float32),
            pltpu.VMEM_SHARED((_NPAD, _D), jnp.float32),
        ],
    )(_scatter_body)
    return f(p8, contrib, row, zm, zd)


def _epilogue_body(eps_ref, den2_ref, msg2_ref, x_ref, sw_ref, w1_ref, b1_ref,
                   w2_ref, b2_ref, g1_ref, be1_ref, g2_ref, be2_ref, acc_ref,
                   out_ref):
    eps = eps_ref[0]
    denom = den2_ref[0, :_N, 0:1] + den2_ref[1, :_N, 0:1]
    msg = msg2_ref[0, :_N, :] + msg2_ref[1, :_N, :]
    messages = msg / jnp.maximum(denom, 1e-37)
    self_t = jnp.dot(x_ref[...], sw_ref[...], preferred_element_type=jnp.float32)
    out = messages + (1.0 + eps) * self_t
    m1 = jnp.mean(out, axis=0, keepdims=True)
    v1 = jnp.mean(jnp.square(out - m1), axis=0, keepdims=True)
    out = g1_ref[...] * (out - m1) / jnp.sqrt(v1 + 1e-3) + be1_ref[...]
    hid = jnp.dot(out, w1_ref[...], preferred_element_type=jnp.float32) + b1_ref[...]
    hid = jnp.where(hid >= 0.0, hid, 0.1 * hid)
    m2 = jnp.mean(hid, axis=0, keepdims=True)
    v2 = jnp.mean(jnp.square(hid - m2), axis=0, keepdims=True)
    hid = g2_ref[...] * (hid - m2) / jnp.sqrt(v2 + 1e-3) + be2_ref[...]
    of = jnp.dot(hid, w2_ref[...], preferred_element_type=jnp.float32) + b2_ref[...]
    of = jnp.where(of > 0.0, of, jnp.exp(of) - 1.0)
    out_ref[...] = acc_ref[...] + of


@jax.jit
def _epilogue(den2, msg2, x, sw, w1, b1, w2, b2, g1, be1, g2, be2, acc, eps):
    return pl.pallas_call(
        _epilogue_body,
        in_specs=[pl.BlockSpec(memory_space=pltpu.SMEM)] + [pl.BlockSpec()] * 13,
        out_specs=pl.BlockSpec(),
        out_shape=jax.ShapeDtypeStruct((_N, _D), jnp.float32),
    )(eps.reshape(1), den2, msg2, x, sw, w1, b1.reshape(1, -1), w2,
      b2.reshape(1, -1), g1.reshape(1, -1), be1.reshape(1, -1),
      g2.reshape(1, -1), be2.reshape(1, -1), acc)


def kernel(x, adj_indices, transform, self_weight, mlp_w1, mlp_b1, mlp_w2,
           mlp_b2, bn1_gamma, bn1_beta, bn2_gamma, bn2_beta, curvature, epsilon):
    row = adj_indices[:, 0]
    col = adj_indices[:, 1]
    zm = jnp.zeros((_STRIPE, _D), dtype=jnp.float32)
    zd = jnp.zeros((_STRIPE, 8), dtype=jnp.float32)
    acc = jnp.zeros((_N, _D), dtype=jnp.float32)
    for h in range(_H):
        th = _xform(x, transform[h], curvature)
        ni, nj = _gather(th, row, col)
        p8, contrib = _edge_chain(ni, nj, curvature)
        den2, msg2 = _scatter(p8, contrib, row, zm, zd)
        acc = _epilogue(den2, msg2, x, self_weight[h], mlp_w1, mlp_b1,
                        mlp_w2, mlp_b2, bn1_gamma, bn1_beta, bn2_gamma,
                        bn2_beta, acc, epsilon)
    output = acc / 8.0
    return (output, curvature)
